# baseline (speedup 1.0000x reference)
import functools

import jax
import jax.numpy as jnp
from jax.experimental import pallas as pl
from jax.experimental.pallas import tpu as pltpu

_SUB = 8


def _transh_kernel(
    pos_ref, neg_ref,
    ent_hbm,
    rel_ref,
    nrm_ref,
    out_ref,
    ent_vmem,
    pht, ptt, nht, ntt,
    prt, pwt, nrt, nwt,
    copy_sem,
    *, margin, alpha, batch, dim, n_rows, cchunk):
  n_groups = n_rows // _SUB

  cp = pltpu.make_async_copy(ent_hbm, ent_vmem, copy_sem)
  cp.start()

  def rel_body(c, carry):
    base = c * (3 * _SUB)
    for u in range(_SUB):
      pr = pos_ref[base + 3 * u + 1]
      nr = neg_ref[base + 3 * u + 1]
      prt[c, u] = rel_ref[pr, 0]
      pwt[c, u] = nrm_ref[pr, 0]
      nrt[c, u] = rel_ref[nr, 0]
      nwt[c, u] = nrm_ref[nr, 0]
    return carry
  jax.lax.fori_loop(0, n_groups, rel_body, 0)

  cp.wait()

  def ent_body(c, carry):
    base = c * (3 * _SUB)
    for u in range(_SUB):
      ph = pos_ref[base + 3 * u]
      pt = pos_ref[base + 3 * u + 2]
      nh = neg_ref[base + 3 * u]
      nt = neg_ref[base + 3 * u + 2]
      pht[c, u] = ent_vmem[ph, 0]
      ptt[c, u] = ent_vmem[pt, 0]
      nht[c, u] = ent_vmem[nh, 0]
      ntt[c, u] = ent_vmem[nt, 0]
    return carry
  jax.lax.fori_loop(0, n_groups, ent_body, 0)

  cgroups = cchunk // _SUB
  n_cchunks = n_rows // cchunk
  inv_dim = 1.0 / dim

  def side(h, r, t, w):
    d = h - t
    dw = jnp.sum(d * w, axis=2, keepdims=True)
    scores = (d + r) - dw * w
    dist = jnp.sum(jnp.abs(scores), axis=2, keepdims=True)
    q = jnp.sum(jnp.abs(h) + jnp.abs(t) + (r * r) * inv_dim,
                axis=2, keepdims=True)
    return dist, q

  def compute_body(c, carry):
    hinge_s, q_s = carry
    sl = pl.ds(c * cgroups, cgroups)
    pd, p_q = side(pht[sl], prt[sl], ptt[sl], pwt[sl])
    nd, n_q = side(nht[sl], nrt[sl], ntt[sl], nwt[sl])

    rows = (c * cchunk
            + _SUB * jax.lax.broadcasted_iota(jnp.int32, (cgroups, _SUB, 1), 0)
            + jax.lax.broadcasted_iota(jnp.int32, (cgroups, _SUB, 1), 1))
    mask = (rows < batch).astype(jnp.float32)
    hinge = jnp.maximum(pd - nd + margin, 0.0)
    return (hinge_s + jnp.sum(hinge * mask),
            q_s + jnp.sum((p_q + n_q) * mask))

  zero = jnp.float32(0.0)
  hinge_s, q_s = jax.lax.fori_loop(
      0, n_cchunks, compute_body, (zero, zero))

  inv_b = 1.0 / batch
  s = (hinge_s * inv_b + (alpha / 3.0) * (q_s * inv_b)
       - 4.0 * alpha / 3.0)
  out_ref[...] = jnp.reshape(s, (1, 1, 1))


def _transh_loss(ent_emb, rel_emb, norm_vec, pos_triplets, neg_triplets,
                 *, margin=4.0, alpha=0.01):
  B = int(pos_triplets.shape[0])
  E, D = int(ent_emb.shape[0]), int(ent_emb.shape[1])
  R = int(rel_emb.shape[0])

  cchunk = 256
  n_rows = pl.cdiv(B, cchunk) * cchunk
  n_groups = n_rows // _SUB

  ent3 = ent_emb.astype(jnp.float32).reshape(E, 1, D)
  rel3 = rel_emb.astype(jnp.float32).reshape(R, 1, D)
  nrm3 = norm_vec.astype(jnp.float32).reshape(R, 1, D)

  def flat(trip):
    f = trip.astype(jnp.int32).reshape(3 * B)
    if n_rows != B:
      f = jnp.pad(f, (0, 3 * (n_rows - B)))
    return f

  pos_flat, neg_flat = flat(pos_triplets), flat(neg_triplets)

  tiles_bytes = n_rows * 8 * D * 4
  vmem_bytes = (E * D + 2 * R * D) * 4 + tiles_bytes + (8 << 20)
  grid_spec = pltpu.PrefetchScalarGridSpec(
      num_scalar_prefetch=2,
      grid=(1,),
      in_specs=[pl.BlockSpec(memory_space=pl.ANY),
                pl.BlockSpec((R, 1, D), lambda c, *_: (0, 0, 0)),
                pl.BlockSpec((R, 1, D), lambda c, *_: (0, 0, 0))],
      out_specs=pl.BlockSpec((1, 1, 1), lambda c, *_: (0, 0, 0)),
      scratch_shapes=[pltpu.VMEM((E, 1, D), jnp.float32)]
                     + [pltpu.VMEM((n_groups, _SUB, D), jnp.float32)] * 8
                     + [pltpu.SemaphoreType.DMA])
  out = pl.pallas_call(
      functools.partial(_transh_kernel, margin=float(margin),
                        alpha=float(alpha), batch=B, dim=D,
                        n_rows=n_rows, cchunk=cchunk),
      out_shape=jax.ShapeDtypeStruct((1, 1, 1), jnp.float32),
      grid_spec=grid_spec,
      compiler_params=pltpu.CompilerParams(
          dimension_semantics=("arbitrary",),
          vmem_limit_bytes=int(min(58 * 2**20, vmem_bytes))),
      cost_estimate=pl.CostEstimate(
          flops=2 * n_rows * D * 30,
          transcendentals=0,
          bytes_accessed=(E * D + 2 * R * D + 4 * n_rows * D
                          + 6 * n_rows) * 4),
      name="transh_loss",
  )(pos_flat, neg_flat, ent3, rel3, nrm3)

  return out[0, 0, 0]


def kernel(ent_emb, rel_emb, norm_vec, pos_triplets, neg_triplets):
  return _transh_loss(ent_emb, rel_emb, norm_vec, pos_triplets, neg_triplets,
                      margin=4.0, alpha=0.01)


# speedup vs baseline: 1.1121x; 1.1121x over previous
import functools

import jax
import jax.numpy as jnp
from jax.experimental import pallas as pl
from jax.experimental.pallas import tpu as pltpu

_SUB = 8
_CCHUNK = 128
_CGRP = _CCHUNK // _SUB


def _transh_kernel(
    pos_ref, neg_ref,
    ent_hbm,
    rel_ref,
    nrm_ref,
    out_ref,
    ent_vmem,
    prt, pwt, nrt, nwt,
    pha, pta, nha, nta,
    phb, ptb, nhb, ntb,
    copy_sem,
    *, margin, alpha, batch, dim, n_rows):
  n_groups = n_rows // _SUB
  n_cchunks = n_rows // _CCHUNK
  inv_dim = 1.0 / dim

  cp = pltpu.make_async_copy(ent_hbm, ent_vmem, copy_sem)
  cp.start()

  def rel_body(c, carry):
    base = c * (3 * _SUB)
    for u in range(_SUB):
      pr = pos_ref[base + 3 * u + 1]
      nr = neg_ref[base + 3 * u + 1]
      prt[c, u] = rel_ref[pr, 0]
      pwt[c, u] = nrm_ref[pr, 0]
      nrt[c, u] = rel_ref[nr, 0]
      nwt[c, u] = nrm_ref[nr, 0]
    return carry
  jax.lax.fori_loop(0, n_groups, rel_body, 0)

  cp.wait()

  def gather_chunk(c, ht, tt, nh_t, nt_t):
    for g in range(_CGRP):
      base = (c * _CGRP + g) * (3 * _SUB)
      for u in range(_SUB):
        ph = pos_ref[base + 3 * u]
        pt = pos_ref[base + 3 * u + 2]
        nh = neg_ref[base + 3 * u]
        nt = neg_ref[base + 3 * u + 2]
        ht[g, u] = ent_vmem[ph, 0]
        tt[g, u] = ent_vmem[pt, 0]
        nh_t[g, u] = ent_vmem[nh, 0]
        nt_t[g, u] = ent_vmem[nt, 0]

  def side(h, r, t, w):
    d = h - t
    dw = jnp.sum(d * w, axis=2, keepdims=True)
    scores = (d + r) - dw * w
    dist = jnp.sum(jnp.abs(scores), axis=2, keepdims=True)
    q = jnp.sum(jnp.abs(h) + jnp.abs(t) + (r * r) * inv_dim,
                axis=2, keepdims=True)
    return dist, q

  def chunk_sums(c, ht, tt, nh_t, nt_t):
    sl = pl.ds(c * _CGRP, _CGRP)
    pd, p_q = side(ht[...], prt[sl], tt[...], pwt[sl])
    nd, n_q = side(nh_t[...], nrt[sl], nt_t[...], nwt[sl])
    rows = (c * _CCHUNK
            + _SUB * jax.lax.broadcasted_iota(jnp.int32, (_CGRP, _SUB, 1), 0)
            + jax.lax.broadcasted_iota(jnp.int32, (_CGRP, _SUB, 1), 1))
    mask = (rows < batch).astype(jnp.float32)
    hinge = jnp.maximum(pd - nd + margin, 0.0)
    return jnp.sum(hinge * mask), jnp.sum((p_q + n_q) * mask)

  def pipe_body(sc, carry):
    hinge_s, q_s = carry
    c0 = 2 * sc
    gather_chunk(c0, pha, pta, nha, nta)
    cg = jnp.maximum(c0 - 1, 0)
    hb, qb = chunk_sums(cg, phb, ptb, nhb, ntb)
    valid = sc > 0
    hinge_s = hinge_s + jnp.where(valid, hb, 0.0)
    q_s = q_s + jnp.where(valid, qb, 0.0)
    gather_chunk(c0 + 1, phb, ptb, nhb, ntb)
    ha, qa = chunk_sums(c0, pha, pta, nha, nta)
    return (hinge_s + ha, q_s + qa)

  zero = jnp.float32(0.0)
  hinge_s, q_s = jax.lax.fori_loop(
      0, n_cchunks // 2, pipe_body, (zero, zero))
  hl, ql = chunk_sums(n_cchunks - 1, phb, ptb, nhb, ntb)
  hinge_s, q_s = hinge_s + hl, q_s + ql

  inv_b = 1.0 / batch
  s = (hinge_s * inv_b + (alpha / 3.0) * (q_s * inv_b)
       - 4.0 * alpha / 3.0)
  out_ref[...] = jnp.reshape(s, (1, 1, 1))


def _transh_loss(ent_emb, rel_emb, norm_vec, pos_triplets, neg_triplets,
                 *, margin=4.0, alpha=0.01):
  B = int(pos_triplets.shape[0])
  E, D = int(ent_emb.shape[0]), int(ent_emb.shape[1])
  R = int(rel_emb.shape[0])

  n_rows = pl.cdiv(B, 2 * _CCHUNK) * 2 * _CCHUNK
  n_groups = n_rows // _SUB

  ent3 = ent_emb.astype(jnp.float32).reshape(E, 1, D)
  rel3 = rel_emb.astype(jnp.float32).reshape(R, 1, D)
  nrm3 = norm_vec.astype(jnp.float32).reshape(R, 1, D)

  def flat(trip):
    f = trip.astype(jnp.int32).reshape(3 * B)
    if n_rows != B:
      f = jnp.pad(f, (0, 3 * (n_rows - B)))
    return f

  pos_flat, neg_flat = flat(pos_triplets), flat(neg_triplets)

  tiles_bytes = (n_rows * 4 + 8 * _CCHUNK) * D * 4
  vmem_bytes = (E * D + 2 * R * D) * 4 + tiles_bytes + (8 << 20)
  grid_spec = pltpu.PrefetchScalarGridSpec(
      num_scalar_prefetch=2,
      grid=(1,),
      in_specs=[pl.BlockSpec(memory_space=pl.ANY),
                pl.BlockSpec((R, 1, D), lambda c, *_: (0, 0, 0)),
                pl.BlockSpec((R, 1, D), lambda c, *_: (0, 0, 0))],
      out_specs=pl.BlockSpec((1, 1, 1), lambda c, *_: (0, 0, 0)),
      scratch_shapes=[pltpu.VMEM((E, 1, D), jnp.float32)]
                     + [pltpu.VMEM((n_groups, _SUB, D), jnp.float32)] * 4
                     + [pltpu.VMEM((_CGRP, _SUB, D), jnp.float32)] * 8
                     + [pltpu.SemaphoreType.DMA])
  out = pl.pallas_call(
      functools.partial(_transh_kernel, margin=float(margin),
                        alpha=float(alpha), batch=B, dim=D, n_rows=n_rows),
      out_shape=jax.ShapeDtypeStruct((1, 1, 1), jnp.float32),
      grid_spec=grid_spec,
      compiler_params=pltpu.CompilerParams(
          dimension_semantics=("arbitrary",),
          vmem_limit_bytes=int(min(58 * 2**20, vmem_bytes))),
      cost_estimate=pl.CostEstimate(
          flops=2 * n_rows * D * 30,
          transcendentals=0,
          bytes_accessed=(E * D + 2 * R * D + 4 * n_rows * D
                          + 6 * n_rows) * 4),
      name="transh_loss",
  )(pos_flat, neg_flat, ent3, rel3, nrm3)

  return out[0, 0, 0]


def kernel(ent_emb, rel_emb, norm_vec, pos_triplets, neg_triplets):
  return _transh_loss(ent_emb, rel_emb, norm_vec, pos_triplets, neg_triplets,
                      margin=4.0, alpha=0.01)


# speedup vs baseline: 1.1792x; 1.0603x over previous
import functools

import jax
import jax.numpy as jnp
from jax.experimental import pallas as pl
from jax.experimental.pallas import tpu as pltpu

_SUB = 8
_CCHUNK = 128
_CGRP = _CCHUNK // _SUB


def _transh_kernel(
    ph_idx, pt_idx, nh_idx, nt_idx, pr_idx, nr_idx,
    ent_hbm,
    rel_ref,
    nrm_ref,
    out_ref,
    ent_vmem,
    prt, pwt, nrt, nwt,
    pha, pta, nha, nta,
    phb, ptb, nhb, ntb,
    copy_sem,
    *, margin, alpha, batch, dim, n_rows):
  n_groups = n_rows // _SUB
  n_cchunks = n_rows // _CCHUNK
  inv_dim = 1.0 / dim

  cp = pltpu.make_async_copy(ent_hbm, ent_vmem, copy_sem)
  cp.start()

  def rel_body(c, carry):
    base = c * _SUB
    for u in range(_SUB):
      gi = base + u
      pr = pr_idx[gi]
      nr = nr_idx[gi]
      prt[c, u] = rel_ref[pr, 0]
      pwt[c, u] = nrm_ref[pr, 0]
      nrt[c, u] = rel_ref[nr, 0]
      nwt[c, u] = nrm_ref[nr, 0]
    return carry
  jax.lax.fori_loop(0, n_groups, rel_body, 0)

  cp.wait()

  def gather_chunk(c, ht, tt, nh_t, nt_t):
    for g in range(_CGRP):
      base = (c * _CGRP + g) * _SUB
      for u in range(_SUB):
        gi = base + u
        ht[g, u] = ent_vmem[ph_idx[gi], 0]
        tt[g, u] = ent_vmem[pt_idx[gi], 0]
        nh_t[g, u] = ent_vmem[nh_idx[gi], 0]
        nt_t[g, u] = ent_vmem[nt_idx[gi], 0]

  def side(h, r, t, w):
    d = h - t
    dw = jnp.sum(d * w, axis=2, keepdims=True)
    scores = (d + r) - dw * w
    dist = jnp.sum(jnp.abs(scores), axis=2, keepdims=True)
    q = jnp.sum(jnp.abs(h) + jnp.abs(t) + (r * r) * inv_dim,
                axis=2, keepdims=True)
    return dist, q

  def chunk_sums(c, ht, tt, nh_t, nt_t):
    sl = pl.ds(c * _CGRP, _CGRP)
    pd, p_q = side(ht[...], prt[sl], tt[...], pwt[sl])
    nd, n_q = side(nh_t[...], nrt[sl], nt_t[...], nwt[sl])
    rows = (c * _CCHUNK
            + _SUB * jax.lax.broadcasted_iota(jnp.int32, (_CGRP, _SUB, 1), 0)
            + jax.lax.broadcasted_iota(jnp.int32, (_CGRP, _SUB, 1), 1))
    mask = (rows < batch).astype(jnp.float32)
    hinge = jnp.maximum(pd - nd + margin, 0.0)
    return jnp.sum(hinge * mask), jnp.sum((p_q + n_q) * mask)

  def pipe_body(sc, carry):
    hinge_s, q_s = carry
    c0 = 2 * sc
    gather_chunk(c0, pha, pta, nha, nta)
    cg = jnp.maximum(c0 - 1, 0)
    hb, qb = chunk_sums(cg, phb, ptb, nhb, ntb)
    valid = sc > 0
    hinge_s = hinge_s + jnp.where(valid, hb, 0.0)
    q_s = q_s + jnp.where(valid, qb, 0.0)
    gather_chunk(c0 + 1, phb, ptb, nhb, ntb)
    ha, qa = chunk_sums(c0, pha, pta, nha, nta)
    return (hinge_s + ha, q_s + qa)

  zero = jnp.float32(0.0)
  hinge_s, q_s = jax.lax.fori_loop(
      0, n_cchunks // 2, pipe_body, (zero, zero))
  hl, ql = chunk_sums(n_cchunks - 1, phb, ptb, nhb, ntb)
  hinge_s, q_s = hinge_s + hl, q_s + ql

  inv_b = 1.0 / batch
  s = (hinge_s * inv_b + (alpha / 3.0) * (q_s * inv_b)
       - 4.0 * alpha / 3.0)
  out_ref[...] = jnp.reshape(s, (1, 1, 1))


def _transh_loss(ent_emb, rel_emb, norm_vec, pos_triplets, neg_triplets,
                 *, margin=4.0, alpha=0.01):
  B = int(pos_triplets.shape[0])
  E, D = int(ent_emb.shape[0]), int(ent_emb.shape[1])
  R = int(rel_emb.shape[0])

  n_rows = pl.cdiv(B, 2 * _CCHUNK) * 2 * _CCHUNK
  n_groups = n_rows // _SUB

  ent3 = ent_emb.astype(jnp.float32).reshape(E, 1, D)
  rel3 = rel_emb.astype(jnp.float32).reshape(R, 1, D)
  nrm3 = norm_vec.astype(jnp.float32).reshape(R, 1, D)

  def col(trip, j):
    c = trip[:, j].astype(jnp.int32)
    return jnp.pad(c, (0, n_rows - B))

  ph, pr, pt = col(pos_triplets, 0), col(pos_triplets, 1), col(pos_triplets, 2)
  nh, nr, nt = col(neg_triplets, 0), col(neg_triplets, 1), col(neg_triplets, 2)

  tiles_bytes = (n_rows * 4 + 8 * _CCHUNK) * D * 4
  vmem_bytes = (E * D + 2 * R * D) * 4 + tiles_bytes + (8 << 20)
  grid_spec = pltpu.PrefetchScalarGridSpec(
      num_scalar_prefetch=6,
      grid=(1,),
      in_specs=[pl.BlockSpec(memory_space=pl.ANY),
                pl.BlockSpec((R, 1, D), lambda c, *_: (0, 0, 0)),
                pl.BlockSpec((R, 1, D), lambda c, *_: (0, 0, 0))],
      out_specs=pl.BlockSpec((1, 1, 1), lambda c, *_: (0, 0, 0)),
      scratch_shapes=[pltpu.VMEM((E, 1, D), jnp.float32)]
                     + [pltpu.VMEM((n_groups, _SUB, D), jnp.float32)] * 4
                     + [pltpu.VMEM((_CGRP, _SUB, D), jnp.float32)] * 8
                     + [pltpu.SemaphoreType.DMA])
  out = pl.pallas_call(
      functools.partial(_transh_kernel, margin=float(margin),
                        alpha=float(alpha), batch=B, dim=D, n_rows=n_rows),
      out_shape=jax.ShapeDtypeStruct((1, 1, 1), jnp.float32),
      grid_spec=grid_spec,
      compiler_params=pltpu.CompilerParams(
          dimension_semantics=("arbitrary",),
          vmem_limit_bytes=int(min(58 * 2**20, vmem_bytes))),
      cost_estimate=pl.CostEstimate(
          flops=2 * n_rows * D * 30,
          transcendentals=0,
          bytes_accessed=(E * D + 2 * R * D + 4 * n_rows * D
                          + 6 * n_rows) * 4),
      name="transh_loss",
  )(ph, pt, nh, nt, pr, nr, ent3, rel3, nrm3)

  return out[0, 0, 0]


def kernel(ent_emb, rel_emb, norm_vec, pos_triplets, neg_triplets):
  return _transh_loss(ent_emb, rel_emb, norm_vec, pos_triplets, neg_triplets,
                      margin=4.0, alpha=0.01)


# speedup vs baseline: 1.2371x; 1.0490x over previous
import functools

import jax
import jax.numpy as jnp
from jax.experimental import pallas as pl
from jax.experimental.pallas import tpu as pltpu

_SUB = 8
_CCHUNK = 256
_CGRP = _CCHUNK // _SUB


def _transh_kernel(
    ph_idx, pt_idx, nh_idx, nt_idx, pr_idx, nr_idx,
    ent_hbm,
    rel_ref,
    nrm_ref,
    out_ref,
    ent_vmem,
    prt, pwt, nrt, nwt,
    pha, pta, nha, nta,
    phb, ptb, nhb, ntb,
    copy_sem,
    *, margin, alpha, batch, dim, n_rows):
  n_groups = n_rows // _SUB
  n_cchunks = n_rows // _CCHUNK
  inv_dim = 1.0 / dim

  cp = pltpu.make_async_copy(ent_hbm, ent_vmem, copy_sem)
  cp.start()

  def rel_body(c, carry):
    base = c * _SUB
    for u in range(_SUB):
      gi = base + u
      pr = pr_idx[gi]
      nr = nr_idx[gi]
      prt[c, u] = rel_ref[pr, 0]
      pwt[c, u] = nrm_ref[pr, 0]
      nrt[c, u] = rel_ref[nr, 0]
      nwt[c, u] = nrm_ref[nr, 0]
    return carry
  jax.lax.fori_loop(0, n_groups, rel_body, 0)

  cp.wait()

  def gather_chunk(c, ht, tt, nh_t, nt_t):
    for g in range(_CGRP):
      base = (c * _CGRP + g) * _SUB
      for u in range(_SUB):
        gi = base + u
        ht[g, u] = ent_vmem[ph_idx[gi], 0]
        tt[g, u] = ent_vmem[pt_idx[gi], 0]
        nh_t[g, u] = ent_vmem[nh_idx[gi], 0]
        nt_t[g, u] = ent_vmem[nt_idx[gi], 0]

  def side(h, r, t, w):
    d = h - t
    dw = jnp.sum(d * w, axis=2, keepdims=True)
    scores = (d + r) - dw * w
    dist = jnp.sum(jnp.abs(scores), axis=2, keepdims=True)
    q = jnp.sum(jnp.abs(h) + jnp.abs(t) + (r * r) * inv_dim,
                axis=2, keepdims=True)
    return dist, q

  def chunk_sums(c, ht, tt, nh_t, nt_t):
    sl = pl.ds(c * _CGRP, _CGRP)
    pd, p_q = side(ht[...], prt[sl], tt[...], pwt[sl])
    nd, n_q = side(nh_t[...], nrt[sl], nt_t[...], nwt[sl])
    rows = (c * _CCHUNK
            + _SUB * jax.lax.broadcasted_iota(jnp.int32, (_CGRP, _SUB, 1), 0)
            + jax.lax.broadcasted_iota(jnp.int32, (_CGRP, _SUB, 1), 1))
    mask = (rows < batch).astype(jnp.float32)
    hinge = jnp.maximum(pd - nd + margin, 0.0)
    return jnp.sum(hinge * mask), jnp.sum((p_q + n_q) * mask)

  def pipe_body(sc, carry):
    hinge_s, q_s = carry
    c0 = 2 * sc
    gather_chunk(c0, pha, pta, nha, nta)
    cg = jnp.maximum(c0 - 1, 0)
    hb, qb = chunk_sums(cg, phb, ptb, nhb, ntb)
    valid = sc > 0
    hinge_s = hinge_s + jnp.where(valid, hb, 0.0)
    q_s = q_s + jnp.where(valid, qb, 0.0)
    gather_chunk(c0 + 1, phb, ptb, nhb, ntb)
    ha, qa = chunk_sums(c0, pha, pta, nha, nta)
    return (hinge_s + ha, q_s + qa)

  zero = jnp.float32(0.0)
  hinge_s, q_s = jax.lax.fori_loop(
      0, n_cchunks // 2, pipe_body, (zero, zero))
  hl, ql = chunk_sums(n_cchunks - 1, phb, ptb, nhb, ntb)
  hinge_s, q_s = hinge_s + hl, q_s + ql

  inv_b = 1.0 / batch
  s = (hinge_s * inv_b + (alpha / 3.0) * (q_s * inv_b)
       - 4.0 * alpha / 3.0)
  out_ref[...] = jnp.reshape(s, (1, 1, 1))


def _transh_loss(ent_emb, rel_emb, norm_vec, pos_triplets, neg_triplets,
                 *, margin=4.0, alpha=0.01):
  B = int(pos_triplets.shape[0])
  E, D = int(ent_emb.shape[0]), int(ent_emb.shape[1])
  R = int(rel_emb.shape[0])

  n_rows = pl.cdiv(B, 2 * _CCHUNK) * 2 * _CCHUNK
  n_groups = n_rows // _SUB

  ent3 = ent_emb.astype(jnp.float32).reshape(E, 1, D)
  rel3 = rel_emb.astype(jnp.float32).reshape(R, 1, D)
  nrm3 = norm_vec.astype(jnp.float32).reshape(R, 1, D)

  def col(trip, j):
    c = trip[:, j].astype(jnp.int32)
    return jnp.pad(c, (0, n_rows - B))

  ph, pr, pt = col(pos_triplets, 0), col(pos_triplets, 1), col(pos_triplets, 2)
  nh, nr, nt = col(neg_triplets, 0), col(neg_triplets, 1), col(neg_triplets, 2)

  tiles_bytes = (n_rows * 4 + 8 * _CCHUNK) * D * 4
  vmem_bytes = (E * D + 2 * R * D) * 4 + tiles_bytes + (8 << 20)
  grid_spec = pltpu.PrefetchScalarGridSpec(
      num_scalar_prefetch=6,
      grid=(1,),
      in_specs=[pl.BlockSpec(memory_space=pl.ANY),
                pl.BlockSpec((R, 1, D), lambda c, *_: (0, 0, 0)),
                pl.BlockSpec((R, 1, D), lambda c, *_: (0, 0, 0))],
      out_specs=pl.BlockSpec((1, 1, 1), lambda c, *_: (0, 0, 0)),
      scratch_shapes=[pltpu.VMEM((E, 1, D), jnp.float32)]
                     + [pltpu.VMEM((n_groups, _SUB, D), jnp.float32)] * 4
                     + [pltpu.VMEM((_CGRP, _SUB, D), jnp.float32)] * 8
                     + [pltpu.SemaphoreType.DMA])
  out = pl.pallas_call(
      functools.partial(_transh_kernel, margin=float(margin),
                        alpha=float(alpha), batch=B, dim=D, n_rows=n_rows),
      out_shape=jax.ShapeDtypeStruct((1, 1, 1), jnp.float32),
      grid_spec=grid_spec,
      compiler_params=pltpu.CompilerParams(
          dimension_semantics=("arbitrary",),
          vmem_limit_bytes=int(min(58 * 2**20, vmem_bytes))),
      cost_estimate=pl.CostEstimate(
          flops=2 * n_rows * D * 30,
          transcendentals=0,
          bytes_accessed=(E * D + 2 * R * D + 4 * n_rows * D
                          + 6 * n_rows) * 4),
      name="transh_loss",
  )(ph, pt, nh, nt, pr, nr, ent3, rel3, nrm3)

  return out[0, 0, 0]


def kernel(ent_emb, rel_emb, norm_vec, pos_triplets, neg_triplets):
  return _transh_loss(ent_emb, rel_emb, norm_vec, pos_triplets, neg_triplets,
                      margin=4.0, alpha=0.01)


# speedup vs baseline: 1.2388x; 1.0014x over previous
import functools

import jax
import jax.numpy as jnp
from jax.experimental import pallas as pl
from jax.experimental.pallas import tpu as pltpu

_SUB = 8
_CCHUNK = 512
_CGRP = _CCHUNK // _SUB


def _transh_kernel(
    ph_idx, pt_idx, nh_idx, nt_idx, pr_idx, nr_idx,
    ent_hbm,
    rel_ref,
    nrm_ref,
    out_ref,
    ent_vmem,
    prt, pwt, nrt, nwt,
    pha, pta, nha, nta,
    phb, ptb, nhb, ntb,
    copy_sem,
    *, margin, alpha, batch, dim, n_rows):
  n_groups = n_rows // _SUB
  n_cchunks = n_rows // _CCHUNK
  inv_dim = 1.0 / dim

  cp = pltpu.make_async_copy(ent_hbm, ent_vmem, copy_sem)
  cp.start()

  def rel_body(c, carry):
    base = c * _SUB
    for u in range(_SUB):
      gi = base + u
      pr = pr_idx[gi]
      nr = nr_idx[gi]
      prt[c, u] = rel_ref[pr, 0]
      pwt[c, u] = nrm_ref[pr, 0]
      nrt[c, u] = rel_ref[nr, 0]
      nwt[c, u] = nrm_ref[nr, 0]
    return carry
  jax.lax.fori_loop(0, n_groups, rel_body, 0)

  cp.wait()

  def gather_chunk(c, ht, tt, nh_t, nt_t):
    for g in range(_CGRP):
      base = (c * _CGRP + g) * _SUB
      for u in range(_SUB):
        gi = base + u
        ht[g, u] = ent_vmem[ph_idx[gi], 0]
        tt[g, u] = ent_vmem[pt_idx[gi], 0]
        nh_t[g, u] = ent_vmem[nh_idx[gi], 0]
        nt_t[g, u] = ent_vmem[nt_idx[gi], 0]

  def side(h, r, t, w):
    d = h - t
    dw = jnp.sum(d * w, axis=2, keepdims=True)
    scores = (d + r) - dw * w
    dist = jnp.sum(jnp.abs(scores), axis=2, keepdims=True)
    q = jnp.sum(jnp.abs(h) + jnp.abs(t) + (r * r) * inv_dim,
                axis=2, keepdims=True)
    return dist, q

  def chunk_sums(c, ht, tt, nh_t, nt_t):
    sl = pl.ds(c * _CGRP, _CGRP)
    pd, p_q = side(ht[...], prt[sl], tt[...], pwt[sl])
    nd, n_q = side(nh_t[...], nrt[sl], nt_t[...], nwt[sl])
    rows = (c * _CCHUNK
            + _SUB * jax.lax.broadcasted_iota(jnp.int32, (_CGRP, _SUB, 1), 0)
            + jax.lax.broadcasted_iota(jnp.int32, (_CGRP, _SUB, 1), 1))
    mask = (rows < batch).astype(jnp.float32)
    hinge = jnp.maximum(pd - nd + margin, 0.0)
    return jnp.sum(hinge * mask), jnp.sum((p_q + n_q) * mask)

  def pipe_body(sc, carry):
    hinge_s, q_s = carry
    c0 = 2 * sc
    gather_chunk(c0, pha, pta, nha, nta)
    cg = jnp.maximum(c0 - 1, 0)
    hb, qb = chunk_sums(cg, phb, ptb, nhb, ntb)
    valid = sc > 0
    hinge_s = hinge_s + jnp.where(valid, hb, 0.0)
    q_s = q_s + jnp.where(valid, qb, 0.0)
    gather_chunk(c0 + 1, phb, ptb, nhb, ntb)
    ha, qa = chunk_sums(c0, pha, pta, nha, nta)
    return (hinge_s + ha, q_s + qa)

  zero = jnp.float32(0.0)
  hinge_s, q_s = jax.lax.fori_loop(
      0, n_cchunks // 2, pipe_body, (zero, zero))
  hl, ql = chunk_sums(n_cchunks - 1, phb, ptb, nhb, ntb)
  hinge_s, q_s = hinge_s + hl, q_s + ql

  inv_b = 1.0 / batch
  s = (hinge_s * inv_b + (alpha / 3.0) * (q_s * inv_b)
       - 4.0 * alpha / 3.0)
  out_ref[...] = jnp.reshape(s, (1, 1, 1))


def _transh_loss(ent_emb, rel_emb, norm_vec, pos_triplets, neg_triplets,
                 *, margin=4.0, alpha=0.01):
  B = int(pos_triplets.shape[0])
  E, D = int(ent_emb.shape[0]), int(ent_emb.shape[1])
  R = int(rel_emb.shape[0])

  n_rows = pl.cdiv(B, 2 * _CCHUNK) * 2 * _CCHUNK
  n_groups = n_rows // _SUB

  ent3 = ent_emb.astype(jnp.float32).reshape(E, 1, D)
  rel3 = rel_emb.astype(jnp.float32).reshape(R, 1, D)
  nrm3 = norm_vec.astype(jnp.float32).reshape(R, 1, D)

  def col(trip, j):
    c = trip[:, j].astype(jnp.int32)
    return jnp.pad(c, (0, n_rows - B))

  ph, pr, pt = col(pos_triplets, 0), col(pos_triplets, 1), col(pos_triplets, 2)
  nh, nr, nt = col(neg_triplets, 0), col(neg_triplets, 1), col(neg_triplets, 2)

  tiles_bytes = (n_rows * 4 + 8 * _CCHUNK) * D * 4
  vmem_bytes = (E * D + 2 * R * D) * 4 + tiles_bytes + (8 << 20)
  grid_spec = pltpu.PrefetchScalarGridSpec(
      num_scalar_prefetch=6,
      grid=(1,),
      in_specs=[pl.BlockSpec(memory_space=pl.ANY),
                pl.BlockSpec((R, 1, D), lambda c, *_: (0, 0, 0)),
                pl.BlockSpec((R, 1, D), lambda c, *_: (0, 0, 0))],
      out_specs=pl.BlockSpec((1, 1, 1), lambda c, *_: (0, 0, 0)),
      scratch_shapes=[pltpu.VMEM((E, 1, D), jnp.float32)]
                     + [pltpu.VMEM((n_groups, _SUB, D), jnp.float32)] * 4
                     + [pltpu.VMEM((_CGRP, _SUB, D), jnp.float32)] * 8
                     + [pltpu.SemaphoreType.DMA])
  out = pl.pallas_call(
      functools.partial(_transh_kernel, margin=float(margin),
                        alpha=float(alpha), batch=B, dim=D, n_rows=n_rows),
      out_shape=jax.ShapeDtypeStruct((1, 1, 1), jnp.float32),
      grid_spec=grid_spec,
      compiler_params=pltpu.CompilerParams(
          dimension_semantics=("arbitrary",),
          vmem_limit_bytes=int(min(58 * 2**20, vmem_bytes))),
      cost_estimate=pl.CostEstimate(
          flops=2 * n_rows * D * 30,
          transcendentals=0,
          bytes_accessed=(E * D + 2 * R * D + 4 * n_rows * D
                          + 6 * n_rows) * 4),
      name="transh_loss",
  )(ph, pt, nh, nt, pr, nr, ent3, rel3, nrm3)

  return out[0, 0, 0]


def kernel(ent_emb, rel_emb, norm_vec, pos_triplets, neg_triplets):
  return _transh_loss(ent_emb, rel_emb, norm_vec, pos_triplets, neg_triplets,
                      margin=4.0, alpha=0.01)


# speedup vs baseline: 1.2407x; 1.0015x over previous
import functools

import jax
import jax.numpy as jnp
from jax.experimental import pallas as pl
from jax.experimental.pallas import tpu as pltpu

_SUB = 8
_CCHUNK = 256
_CGRP = _CCHUNK // _SUB


def _transh_kernel(
    ph_idx, pt_idx, nh_idx, nt_idx, pr_idx, nr_idx,
    ent_hbm,
    rel_ref,
    nrm_ref,
    out_ref,
    ent_vmem,
    prt, pwt, nrt, nwt,
    pha, pta, nha, nta,
    phb, ptb, nhb, ntb,
    copy_sem,
    *, margin, alpha, batch, dim, n_rows):
  n_groups = n_rows // _SUB
  n_cchunks = n_rows // _CCHUNK
  inv_dim = 1.0 / dim

  cp = pltpu.make_async_copy(ent_hbm, ent_vmem, copy_sem)
  cp.start()

  def rel_body(c, carry):
    base = c * _SUB
    for u in range(_SUB):
      gi = base + u
      pr = pr_idx[gi]
      nr = nr_idx[gi]
      prt[c, u] = rel_ref[pr, 0]
      pwt[c, u] = nrm_ref[pr, 0]
      nrt[c, u] = rel_ref[nr, 0]
      nwt[c, u] = nrm_ref[nr, 0]
    return carry
  jax.lax.fori_loop(0, n_groups, rel_body, 0)

  cp.wait()

  def gather_chunk(c, ht, tt, nh_t, nt_t):
    for g in range(_CGRP):
      base = (c * _CGRP + g) * _SUB
      for u in range(_SUB):
        gi = base + u
        ht[g, u] = ent_vmem[ph_idx[gi], 0]
        tt[g, u] = ent_vmem[pt_idx[gi], 0]
        nh_t[g, u] = ent_vmem[nh_idx[gi], 0]
        nt_t[g, u] = ent_vmem[nt_idx[gi], 0]

  def side(h, r, t, w):
    d = h - t
    dw = jnp.sum(d * w, axis=2, keepdims=True)
    scores = (d + r) - dw * w
    dist = jnp.sum(jnp.abs(scores), axis=2, keepdims=True)
    q = jnp.sum(jnp.abs(h) + jnp.abs(t) + (r * r) * inv_dim,
                axis=2, keepdims=True)
    return dist, q

  def chunk_sums(c, ht, tt, nh_t, nt_t):
    sl = pl.ds(c * _CGRP, _CGRP)
    pd, p_q = side(ht[...], prt[sl], tt[...], pwt[sl])
    nd, n_q = side(nh_t[...], nrt[sl], nt_t[...], nwt[sl])
    rows = (c * _CCHUNK
            + _SUB * jax.lax.broadcasted_iota(jnp.int32, (_CGRP, _SUB, 1), 0)
            + jax.lax.broadcasted_iota(jnp.int32, (_CGRP, _SUB, 1), 1))
    mask = (rows < batch).astype(jnp.float32)
    hinge = jnp.maximum(pd - nd + margin, 0.0)
    return jnp.sum(hinge * mask), jnp.sum((p_q + n_q) * mask)

  gather_chunk(0, pha, pta, nha, nta)

  def pipe_body(sc, carry):
    hinge_s, q_s = carry
    c0 = 2 * sc
    c1 = c0 + 1
    gather_chunk(c1, phb, ptb, nhb, ntb)
    h0, q0 = chunk_sums(c0, pha, pta, nha, nta)
    cnext = jnp.minimum(c0 + 2, n_cchunks - 1)
    gather_chunk(cnext, pha, pta, nha, nta)
    h1, q1 = chunk_sums(c1, phb, ptb, nhb, ntb)
    return (hinge_s + h0 + h1, q_s + q0 + q1)

  zero = jnp.float32(0.0)
  hinge_s, q_s = jax.lax.fori_loop(
      0, n_cchunks // 2, pipe_body, (zero, zero))

  inv_b = 1.0 / batch
  s = (hinge_s * inv_b + (alpha / 3.0) * (q_s * inv_b)
       - 4.0 * alpha / 3.0)
  out_ref[...] = jnp.reshape(s, (1, 1, 1))


def _transh_loss(ent_emb, rel_emb, norm_vec, pos_triplets, neg_triplets,
                 *, margin=4.0, alpha=0.01):
  B = int(pos_triplets.shape[0])
  E, D = int(ent_emb.shape[0]), int(ent_emb.shape[1])
  R = int(rel_emb.shape[0])

  n_rows = pl.cdiv(B, 2 * _CCHUNK) * 2 * _CCHUNK
  n_groups = n_rows // _SUB

  ent3 = ent_emb.astype(jnp.float32).reshape(E, 1, D)
  rel3 = rel_emb.astype(jnp.float32).reshape(R, 1, D)
  nrm3 = norm_vec.astype(jnp.float32).reshape(R, 1, D)

  def col(trip, j):
    c = trip[:, j].astype(jnp.int32)
    return jnp.pad(c, (0, n_rows - B))

  ph, pr, pt = col(pos_triplets, 0), col(pos_triplets, 1), col(pos_triplets, 2)
  nh, nr, nt = col(neg_triplets, 0), col(neg_triplets, 1), col(neg_triplets, 2)

  tiles_bytes = (n_rows * 4 + 8 * _CCHUNK) * D * 4
  vmem_bytes = (E * D + 2 * R * D) * 4 + tiles_bytes + (8 << 20)
  grid_spec = pltpu.PrefetchScalarGridSpec(
      num_scalar_prefetch=6,
      grid=(1,),
      in_specs=[pl.BlockSpec(memory_space=pl.ANY),
                pl.BlockSpec((R, 1, D), lambda c, *_: (0, 0, 0)),
                pl.BlockSpec((R, 1, D), lambda c, *_: (0, 0, 0))],
      out_specs=pl.BlockSpec((1, 1, 1), lambda c, *_: (0, 0, 0)),
      scratch_shapes=[pltpu.VMEM((E, 1, D), jnp.float32)]
                     + [pltpu.VMEM((n_groups, _SUB, D), jnp.float32)] * 4
                     + [pltpu.VMEM((_CGRP, _SUB, D), jnp.float32)] * 8
                     + [pltpu.SemaphoreType.DMA])
  out = pl.pallas_call(
      functools.partial(_transh_kernel, margin=float(margin),
                        alpha=float(alpha), batch=B, dim=D, n_rows=n_rows),
      out_shape=jax.ShapeDtypeStruct((1, 1, 1), jnp.float32),
      grid_spec=grid_spec,
      compiler_params=pltpu.CompilerParams(
          dimension_semantics=("arbitrary",),
          vmem_limit_bytes=int(min(58 * 2**20, vmem_bytes))),
      cost_estimate=pl.CostEstimate(
          flops=2 * n_rows * D * 30,
          transcendentals=0,
          bytes_accessed=(E * D + 2 * R * D + 4 * n_rows * D
                          + 6 * n_rows) * 4),
      name="transh_loss",
  )(ph, pt, nh, nt, pr, nr, ent3, rel3, nrm3)

  return out[0, 0, 0]


def kernel(ent_emb, rel_emb, norm_vec, pos_triplets, neg_triplets):
  return _transh_loss(ent_emb, rel_emb, norm_vec, pos_triplets, neg_triplets,
                      margin=4.0, alpha=0.01)
